# Initial kernel scaffold; baseline (speedup 1.0000x reference)
import jax
import jax.numpy as jnp
from jax.experimental import pallas as pl


def kernel(x_orig, edge_index, missing_mask_tensor, fill_vec, Wq0, bq0, Wk0, bk0, Wv0, bv0, Ws0, bs0, Wb0, g0, be0, Wq1, bq1, Wk1, bk1, Wv1, bv1, Ws1, bs1, Wb1, g1, be1):
    raise NotImplementedError("write your pallas kernel here")


# speedup vs baseline: 46.4373x; 46.4373x over previous
import functools

import jax
import jax.numpy as jnp
from jax import lax
from jax.experimental import pallas as pl
from jax.experimental.pallas import tpu as pltpu
from jax.experimental.pallas import tpu_sc as plsc

N = 10000
E = 320000
D = 128
H = 8
C = 16
HC = 128

NC = 2
NS = 16
NW = NC * NS
EB = 80
EPW = E // NW
NBLK = EPW // EB
NP = 10112
RPS = NP // NS

_NBLK_TC = 10
_BN = N // _NBLK_TC


def _proj0_body(x_ref, m_ref, fill_ref, w_ref, wrow_ref, b_ref, sel_ref,
                qkvs_ref, nrm_ref):
    i = pl.program_id(0)
    x = x_ref[...]
    m = m_ref[...]
    x0 = jnp.where(m > 0.5, fill_ref[...], x)
    acc = jnp.dot(x0, w_ref[...], preferred_element_type=jnp.float32)
    acc = acc + m[:, 0:1] * wrow_ref[...]
    acc = acc + b_ref[...]
    qkvs_ref[...] = acc
    qk = acc[:, :256]
    n2 = jnp.dot(qk * qk, sel_ref[...], preferred_element_type=jnp.float32)
    bmax = jnp.max(n2, axis=0, keepdims=True)

    @pl.when(i == 0)
    def _():
        nrm_ref[...] = bmax

    @pl.when(i > 0)
    def _():
        nrm_ref[...] = jnp.maximum(nrm_ref[...], bmax)


def _proj1_body(x_ref, w_ref, b_ref, sel_ref, qkvs_ref, nrm_ref):
    i = pl.program_id(0)
    x = x_ref[...]
    acc = jnp.dot(x, w_ref[...], preferred_element_type=jnp.float32)
    acc = acc + b_ref[...]
    qkvs_ref[...] = acc
    qk = acc[:, :256]
    n2 = jnp.dot(qk * qk, sel_ref[...], preferred_element_type=jnp.float32)
    bmax = jnp.max(n2, axis=0, keepdims=True)

    @pl.when(i == 0)
    def _():
        nrm_ref[...] = bmax

    @pl.when(i > 0)
    def _():
        nrm_ref[...] = jnp.maximum(nrm_ref[...], bmax)


def _proj0_call(x, m128, fill, w, wrow, b, sel):
    return pl.pallas_call(
        _proj0_body,
        grid=(_NBLK_TC,),
        in_specs=[
            pl.BlockSpec((_BN, 128), lambda i: (i, 0)),
            pl.BlockSpec((_BN, 128), lambda i: (i, 0)),
            pl.BlockSpec((1, 128), lambda i: (0, 0)),
            pl.BlockSpec((128, 512), lambda i: (0, 0)),
            pl.BlockSpec((1, 512), lambda i: (0, 0)),
            pl.BlockSpec((1, 512), lambda i: (0, 0)),
            pl.BlockSpec((256, 16), lambda i: (0, 0)),
        ],
        out_specs=[
            pl.BlockSpec((_BN, 512), lambda i: (i, 0)),
            pl.BlockSpec((1, 16), lambda i: (0, 0)),
        ],
        out_shape=[
            jax.ShapeDtypeStruct((N, 512), jnp.float32),
            jax.ShapeDtypeStruct((1, 16), jnp.float32),
        ],
    )(x, m128, fill, w, wrow, b, sel)


def _proj1_call(x, w, b, sel):
    return pl.pallas_call(
        _proj1_body,
        grid=(_NBLK_TC,),
        in_specs=[
            pl.BlockSpec((_BN, 128), lambda i: (i, 0)),
            pl.BlockSpec((128, 512), lambda i: (0, 0)),
            pl.BlockSpec((1, 512), lambda i: (0, 0)),
            pl.BlockSpec((256, 16), lambda i: (0, 0)),
        ],
        out_specs=[
            pl.BlockSpec((_BN, 512), lambda i: (i, 0)),
            pl.BlockSpec((1, 16), lambda i: (0, 0)),
        ],
        out_shape=[
            jax.ShapeDtypeStruct((N, 512), jnp.float32),
            jax.ShapeDtypeStruct((1, 16), jnp.float32),
        ],
    )(x, w, b, sel)


def _edge_body(q_hbm, k_hbm, v_hbm, dst_hbm, src_hbm, m_hbm, z128_hbm, z16_hbm,
               num_out, den_out,
               num_sh, den_sh, dstv, srcv, qrows, krows, vrows, exv, mv,
               sem0, sem1, sem2):
    wv = qrows
    c = lax.axis_index("c")
    s = lax.axis_index("s")
    wid = c * NS + s

    pltpu.sync_copy(z128_hbm.at[pl.ds(s * RPS, RPS)],
                    num_sh.at[pl.ds(s * RPS, RPS)])
    pltpu.sync_copy(z16_hbm.at[pl.ds(s * RPS, RPS)],
                    den_sh.at[pl.ds(s * RPS, RPS)])
    pltpu.sync_copy(m_hbm, mv)

    plsc.subcore_barrier()

    mvec = mv[...]
    lidx = lax.iota(jnp.int32, 16)
    lane8 = lidx < 8
    ohs = [(lidx == h).astype(jnp.float32) for h in range(H)]
    ebase = wid * EPW

    def blk(i, _):
        b0 = ebase + i * EB
        pltpu.sync_copy(dst_hbm.at[pl.ds(b0, EB)], dstv)
        pltpu.sync_copy(src_hbm.at[pl.ds(b0, EB)], srcv)
        cq = pltpu.async_copy(q_hbm.at[dstv], qrows, sem0)
        ck = pltpu.async_copy(k_hbm.at[srcv], krows, sem1)
        cv = pltpu.async_copy(v_hbm.at[srcv], vrows, sem2)
        cq.wait()
        ck.wait()
        cv.wait()

        def edge(e, _):
            svec = jnp.zeros((16,), jnp.float32)
            for h in range(H):
                ph = qrows[e, pl.ds(h * 16, 16)] * krows[e, pl.ds(h * 16, 16)]
                svec = svec + ohs[h] * jnp.sum(ph)
            ex = jnp.exp(svec * 0.25 - mvec)
            ex = jnp.where(lane8, ex, 0.0)
            exv[e, :] = ex
            for h in range(H):
                bh = jnp.full((16,), ex[h], jnp.float32)
                wv[e, pl.ds(h * 16, 16)] = vrows[e, pl.ds(h * 16, 16)] * bh
            return 0

        lax.fori_loop(0, EB, edge, 0)

        pltpu.sync_copy(wv, num_sh.at[dstv], add=True)
        pltpu.sync_copy(exv, den_sh.at[dstv], add=True)
        return 0

    lax.fori_loop(0, NBLK, blk, 0)

    plsc.subcore_barrier()

    pltpu.sync_copy(num_sh.at[pl.ds(s * RPS, RPS)],
                    num_out.at[c, pl.ds(s * RPS, RPS)])
    pltpu.sync_copy(den_sh.at[pl.ds(s * RPS, RPS)],
                    den_out.at[c, pl.ds(s * RPS, RPS)])


def _edge_call(q, k, v, dst, src, m16, z128, z16):
    mesh = plsc.VectorSubcoreMesh(core_axis_name="c", subcore_axis_name="s")
    kfn = pl.kernel(
        _edge_body,
        out_type=[
            jax.ShapeDtypeStruct((NC, NP, 128), jnp.float32),
            jax.ShapeDtypeStruct((NC, NP, 16), jnp.float32),
        ],
        mesh=mesh,
        compiler_params=pltpu.CompilerParams(needs_layout_passes=False,
                                             use_tc_tiling_on_sc=False),
        scratch_types=[
            pltpu.VMEM_SHARED((NP, 128), jnp.float32),
            pltpu.VMEM_SHARED((NP, 16), jnp.float32),
            pltpu.VMEM((EB,), jnp.int32),
            pltpu.VMEM((EB,), jnp.int32),
            pltpu.VMEM((EB, 128), jnp.float32),
            pltpu.VMEM((EB, 128), jnp.float32),
            pltpu.VMEM((EB, 128), jnp.float32),
            pltpu.VMEM((EB, 16), jnp.float32),
            pltpu.VMEM((16,), jnp.float32),
            pltpu.SemaphoreType.DMA,
            pltpu.SemaphoreType.DMA,
            pltpu.SemaphoreType.DMA,
        ],
    )
    return kfn(q, k, v, dst, src, m16, z128, z16)


def _comb_body(relu, num0_ref, num1_ref, den0_ref, den1_ref, xr_ref,
               wba_ref, wbb_ref, ex_ref, g_ref, be_ref, y_ref):
    den = den0_ref[...] + den1_ref[...]
    den128 = jnp.dot(den, ex_ref[...], preferred_element_type=jnp.float32)
    num = num0_ref[...] + num1_ref[...]
    safe = jnp.where(den128 > 0.0, den128, 1.0)
    out = jnp.where(den128 > 0.0, num / safe, 0.0)
    xr = xr_ref[...]
    bl = jnp.sum(out * wba_ref[...] + xr * wbb_ref[...], axis=1, keepdims=True)
    beta = 1.0 / (1.0 + jnp.exp(-bl))
    y = beta * xr + (1.0 - beta) * out
    mu = jnp.mean(y, axis=1, keepdims=True)
    var = jnp.mean((y - mu) ** 2, axis=1, keepdims=True)
    yn = (y - mu) / jnp.sqrt(var + 1e-5) * g_ref[...] + be_ref[...]
    if relu:
        yn = jnp.maximum(yn, 0.0)
    y_ref[...] = yn


def _comb_call(relu, num0, num1, den0, den1, xr, wba, wbb, exmat, g, be):
    return pl.pallas_call(
        functools.partial(_comb_body, relu),
        grid=(_NBLK_TC,),
        in_specs=[
            pl.BlockSpec((_BN, 128), lambda i: (i, 0)),
            pl.BlockSpec((_BN, 128), lambda i: (i, 0)),
            pl.BlockSpec((_BN, 16), lambda i: (i, 0)),
            pl.BlockSpec((_BN, 16), lambda i: (i, 0)),
            pl.BlockSpec((_BN, 128), lambda i: (i, 0)),
            pl.BlockSpec((1, 128), lambda i: (0, 0)),
            pl.BlockSpec((1, 128), lambda i: (0, 0)),
            pl.BlockSpec((16, 128), lambda i: (0, 0)),
            pl.BlockSpec((1, 128), lambda i: (0, 0)),
            pl.BlockSpec((1, 128), lambda i: (0, 0)),
        ],
        out_specs=pl.BlockSpec((_BN, 128), lambda i: (i, 0)),
        out_shape=jax.ShapeDtypeStruct((N, 128), jnp.float32),
    )(num0, num1, den0, den1, xr, wba, wbb, exmat, g, be)


def _layer(x, src, dst, wcat, wrow, bcat, wba, wbb, g, be, sel, exmat,
           z128, z16, m128, fill, relu):
    if m128 is None:
        qkvs, nrm2 = _proj1_call(x, wcat, bcat, sel)
    else:
        qkvs, nrm2 = _proj0_call(x, m128, fill, wcat, wrow, bcat, sel)
    q = qkvs[:, :128]
    k = qkvs[:, 128:256]
    v = qkvs[:, 256:384]
    xr = qkvs[:, 384:]
    m8 = jnp.sqrt(nrm2[0, :8]) * jnp.sqrt(nrm2[0, 8:]) * 0.25
    m16 = jnp.concatenate([m8, jnp.zeros((8,), jnp.float32)])
    num, den = _edge_call(q, k, v, dst, src, m16, z128, z16)
    num = num[:, :N]
    den = den[:, :N]
    return _comb_call(relu, num[0], num[1], den[0], den[1], xr,
                      wba, wbb, exmat, g, be)


def kernel(x_orig, edge_index, missing_mask_tensor, fill_vec,
           Wq0, bq0, Wk0, bk0, Wv0, bv0, Ws0, bs0, Wb0, g0, be0,
           Wq1, bq1, Wk1, bk1, Wv1, bv1, Ws1, bs1, Wb1, g1, be1):
    src = edge_index[0].astype(jnp.int32)
    dst = edge_index[1].astype(jnp.int32)

    sel = (jnp.arange(256)[:, None] // 16 == jnp.arange(16)[None, :]
           ).astype(jnp.float32)
    exmat = (jnp.arange(16)[:, None] == jnp.arange(128)[None, :] // 16
             ).astype(jnp.float32)
    z128 = jnp.zeros((NP, 128), jnp.float32)
    z16 = jnp.zeros((NP, 16), jnp.float32)
    m128 = jnp.broadcast_to(missing_mask_tensor, (N, 128))

    w0 = jnp.concatenate([Wq0, Wk0, Wv0, Ws0], axis=1)
    b0 = jnp.concatenate([bq0, bk0, bv0, bs0]).reshape(1, 512)
    wba0 = (Wb0[:128, 0] + Wb0[256:, 0]).reshape(1, 128)
    wbb0 = (Wb0[128:256, 0] - Wb0[256:, 0]).reshape(1, 128)

    w1 = jnp.concatenate([Wq1, Wk1, Wv1, Ws1], axis=1)
    b1 = jnp.concatenate([bq1, bk1, bv1, bs1]).reshape(1, 512)
    wba1 = (Wb1[:128, 0] + Wb1[256:, 0]).reshape(1, 128)
    wbb1 = (Wb1[128:256, 0] - Wb1[256:, 0]).reshape(1, 128)

    h = _layer(x_orig, src, dst, w0[:128], w0[128:129].reshape(1, 512), b0,
               wba0, wbb0, g0.reshape(1, 128), be0.reshape(1, 128),
               sel, exmat, z128, z16, m128, fill_vec, True)
    h = _layer(h, src, dst, w1, None, b1,
               wba1, wbb1, g1.reshape(1, 128), be1.reshape(1, 128),
               sel, exmat, z128, z16, None, None, False)
    return h


# speedup vs baseline: 63.8103x; 1.3741x over previous
import functools

import jax
import jax.numpy as jnp
from jax import lax
from jax.experimental import pallas as pl
from jax.experimental.pallas import tpu as pltpu
from jax.experimental.pallas import tpu_sc as plsc

N = 10000
E = 320000
D = 128
H = 8
C = 16
HC = 128

NC = 2
NS = 16
NW = NC * NS
EB = 80
EPW = E // NW
NBLK = EPW // EB
NP = 10112
RPS = NP // NS

_NBLK_TC = 10
_BN = N // _NBLK_TC


def _proj0_body(x_ref, m_ref, fill_ref, w_ref, wrow_ref, b_ref, sel_ref,
                qkvs_ref, nrm_ref):
    i = pl.program_id(0)
    x = x_ref[...]
    m = m_ref[...]
    x0 = jnp.where(m > 0.5, fill_ref[...], x)
    acc = jnp.dot(x0, w_ref[...], preferred_element_type=jnp.float32)
    acc = acc + m[:, 0:1] * wrow_ref[...]
    acc = acc + b_ref[...]
    qkvs_ref[...] = acc
    qk = acc[:, :256]
    n2 = jnp.dot(qk * qk, sel_ref[...], preferred_element_type=jnp.float32)
    bmax = jnp.max(n2, axis=0, keepdims=True)

    @pl.when(i == 0)
    def _():
        nrm_ref[...] = bmax

    @pl.when(i > 0)
    def _():
        nrm_ref[...] = jnp.maximum(nrm_ref[...], bmax)


def _proj1_body(x_ref, w_ref, b_ref, sel_ref, qkvs_ref, nrm_ref):
    i = pl.program_id(0)
    x = x_ref[...]
    acc = jnp.dot(x, w_ref[...], preferred_element_type=jnp.float32)
    acc = acc + b_ref[...]
    qkvs_ref[...] = acc
    qk = acc[:, :256]
    n2 = jnp.dot(qk * qk, sel_ref[...], preferred_element_type=jnp.float32)
    bmax = jnp.max(n2, axis=0, keepdims=True)

    @pl.when(i == 0)
    def _():
        nrm_ref[...] = bmax

    @pl.when(i > 0)
    def _():
        nrm_ref[...] = jnp.maximum(nrm_ref[...], bmax)


def _proj0_call(x, m128, fill, w, wrow, b, sel):
    return pl.pallas_call(
        _proj0_body,
        grid=(_NBLK_TC,),
        in_specs=[
            pl.BlockSpec((_BN, 128), lambda i: (i, 0)),
            pl.BlockSpec((_BN, 128), lambda i: (i, 0)),
            pl.BlockSpec((1, 128), lambda i: (0, 0)),
            pl.BlockSpec((128, 512), lambda i: (0, 0)),
            pl.BlockSpec((1, 512), lambda i: (0, 0)),
            pl.BlockSpec((1, 512), lambda i: (0, 0)),
            pl.BlockSpec((256, 16), lambda i: (0, 0)),
        ],
        out_specs=[
            pl.BlockSpec((_BN, 512), lambda i: (i, 0)),
            pl.BlockSpec((1, 16), lambda i: (0, 0)),
        ],
        out_shape=[
            jax.ShapeDtypeStruct((N, 512), jnp.float32),
            jax.ShapeDtypeStruct((1, 16), jnp.float32),
        ],
    )(x, m128, fill, w, wrow, b, sel)


def _proj1_call(x, w, b, sel):
    return pl.pallas_call(
        _proj1_body,
        grid=(_NBLK_TC,),
        in_specs=[
            pl.BlockSpec((_BN, 128), lambda i: (i, 0)),
            pl.BlockSpec((128, 512), lambda i: (0, 0)),
            pl.BlockSpec((1, 512), lambda i: (0, 0)),
            pl.BlockSpec((256, 16), lambda i: (0, 0)),
        ],
        out_specs=[
            pl.BlockSpec((_BN, 512), lambda i: (i, 0)),
            pl.BlockSpec((1, 16), lambda i: (0, 0)),
        ],
        out_shape=[
            jax.ShapeDtypeStruct((N, 512), jnp.float32),
            jax.ShapeDtypeStruct((1, 16), jnp.float32),
        ],
    )(x, w, b, sel)


def _edge_body(q_hbm, k_hbm, v_hbm, dst_hbm, src_hbm, m_hbm, z128_hbm, z16_hbm,
               num_out, den_out,
               num_sh, den_sh, dstv, srcv, qrows, krows, vrows, exv, mv,
               sem0, sem1, sem2):
    wv = qrows
    c = lax.axis_index("c")
    s = lax.axis_index("s")
    wid = c * NS + s

    pltpu.sync_copy(z128_hbm.at[pl.ds(s * RPS, RPS)],
                    num_sh.at[pl.ds(s * RPS, RPS)])
    pltpu.sync_copy(z16_hbm.at[pl.ds(s * RPS, RPS)],
                    den_sh.at[pl.ds(s * RPS, RPS)])
    pltpu.sync_copy(m_hbm, mv)

    plsc.subcore_barrier()

    mvec = mv[...]
    lidx = lax.iota(jnp.int32, 16)
    lane8 = lidx < 8
    ohs = [(lidx == h).astype(jnp.float32) for h in range(H)]
    ebase = wid * EPW

    def blk(i, _):
        b0 = ebase + i * EB
        pltpu.sync_copy(dst_hbm.at[pl.ds(b0, EB)], dstv)
        pltpu.sync_copy(src_hbm.at[pl.ds(b0, EB)], srcv)
        cq = pltpu.async_copy(q_hbm.at[dstv], qrows, sem0)
        ck = pltpu.async_copy(k_hbm.at[srcv], krows, sem1)
        cv = pltpu.async_copy(v_hbm.at[srcv], vrows, sem2)
        cq.wait()
        ck.wait()
        cv.wait()

        @plsc.parallel_loop(0, EB, step=1, unroll=4)
        def _(e):
            svec = jnp.zeros((16,), jnp.float32)
            for h in range(H):
                ph = qrows[e, pl.ds(h * 16, 16)] * krows[e, pl.ds(h * 16, 16)]
                svec = svec + ohs[h] * jnp.sum(ph)
            ex = jnp.exp(svec * 0.25 - mvec)
            ex = jnp.where(lane8, ex, 0.0)
            exv[e, :] = ex
            for h in range(H):
                bh = jnp.full((16,), ex[h], jnp.float32)
                wv[e, pl.ds(h * 16, 16)] = vrows[e, pl.ds(h * 16, 16)] * bh

        pltpu.sync_copy(wv, num_sh.at[dstv], add=True)
        pltpu.sync_copy(exv, den_sh.at[dstv], add=True)
        return 0

    lax.fori_loop(0, NBLK, blk, 0)

    plsc.subcore_barrier()

    pltpu.sync_copy(num_sh.at[pl.ds(s * RPS, RPS)],
                    num_out.at[c, pl.ds(s * RPS, RPS)])
    pltpu.sync_copy(den_sh.at[pl.ds(s * RPS, RPS)],
                    den_out.at[c, pl.ds(s * RPS, RPS)])


def _edge_call(q, k, v, dst, src, m16, z128, z16):
    mesh = plsc.VectorSubcoreMesh(core_axis_name="c", subcore_axis_name="s")
    kfn = pl.kernel(
        _edge_body,
        out_type=[
            jax.ShapeDtypeStruct((NC, NP, 128), jnp.float32),
            jax.ShapeDtypeStruct((NC, NP, 16), jnp.float32),
        ],
        mesh=mesh,
        compiler_params=pltpu.CompilerParams(needs_layout_passes=False,
                                             use_tc_tiling_on_sc=False),
        scratch_types=[
            pltpu.VMEM_SHARED((NP, 128), jnp.float32),
            pltpu.VMEM_SHARED((NP, 16), jnp.float32),
            pltpu.VMEM((EB,), jnp.int32),
            pltpu.VMEM((EB,), jnp.int32),
            pltpu.VMEM((EB, 128), jnp.float32),
            pltpu.VMEM((EB, 128), jnp.float32),
            pltpu.VMEM((EB, 128), jnp.float32),
            pltpu.VMEM((EB, 16), jnp.float32),
            pltpu.VMEM((16,), jnp.float32),
            pltpu.SemaphoreType.DMA,
            pltpu.SemaphoreType.DMA,
            pltpu.SemaphoreType.DMA,
        ],
    )
    return kfn(q, k, v, dst, src, m16, z128, z16)


def _comb_body(relu, num0_ref, num1_ref, den0_ref, den1_ref, xr_ref,
               wba_ref, wbb_ref, ex_ref, g_ref, be_ref, y_ref):
    den = den0_ref[...] + den1_ref[...]
    den128 = jnp.dot(den, ex_ref[...], preferred_element_type=jnp.float32)
    num = num0_ref[...] + num1_ref[...]
    safe = jnp.where(den128 > 0.0, den128, 1.0)
    out = jnp.where(den128 > 0.0, num / safe, 0.0)
    xr = xr_ref[...]
    bl = jnp.sum(out * wba_ref[...] + xr * wbb_ref[...], axis=1, keepdims=True)
    beta = 1.0 / (1.0 + jnp.exp(-bl))
    y = beta * xr + (1.0 - beta) * out
    mu = jnp.mean(y, axis=1, keepdims=True)
    var = jnp.mean((y - mu) ** 2, axis=1, keepdims=True)
    yn = (y - mu) / jnp.sqrt(var + 1e-5) * g_ref[...] + be_ref[...]
    if relu:
        yn = jnp.maximum(yn, 0.0)
    y_ref[...] = yn


def _comb_call(relu, num0, num1, den0, den1, xr, wba, wbb, exmat, g, be):
    return pl.pallas_call(
        functools.partial(_comb_body, relu),
        grid=(_NBLK_TC,),
        in_specs=[
            pl.BlockSpec((_BN, 128), lambda i: (i, 0)),
            pl.BlockSpec((_BN, 128), lambda i: (i, 0)),
            pl.BlockSpec((_BN, 16), lambda i: (i, 0)),
            pl.BlockSpec((_BN, 16), lambda i: (i, 0)),
            pl.BlockSpec((_BN, 128), lambda i: (i, 0)),
            pl.BlockSpec((1, 128), lambda i: (0, 0)),
            pl.BlockSpec((1, 128), lambda i: (0, 0)),
            pl.BlockSpec((16, 128), lambda i: (0, 0)),
            pl.BlockSpec((1, 128), lambda i: (0, 0)),
            pl.BlockSpec((1, 128), lambda i: (0, 0)),
        ],
        out_specs=pl.BlockSpec((_BN, 128), lambda i: (i, 0)),
        out_shape=jax.ShapeDtypeStruct((N, 128), jnp.float32),
    )(num0, num1, den0, den1, xr, wba, wbb, exmat, g, be)


def _layer(x, src, dst, wcat, wrow, bcat, wba, wbb, g, be, sel, exmat,
           z128, z16, m128, fill, relu):
    if m128 is None:
        qkvs, nrm2 = _proj1_call(x, wcat, bcat, sel)
    else:
        qkvs, nrm2 = _proj0_call(x, m128, fill, wcat, wrow, bcat, sel)
    q = qkvs[:, :128]
    k = qkvs[:, 128:256]
    v = qkvs[:, 256:384]
    xr = qkvs[:, 384:]
    m8 = jnp.sqrt(nrm2[0, :8]) * jnp.sqrt(nrm2[0, 8:]) * 0.25
    m16 = jnp.concatenate([m8, jnp.zeros((8,), jnp.float32)])
    num, den = _edge_call(q, k, v, dst, src, m16, z128, z16)
    num = num[:, :N]
    den = den[:, :N]
    return _comb_call(relu, num[0], num[1], den[0], den[1], xr,
                      wba, wbb, exmat, g, be)


def kernel(x_orig, edge_index, missing_mask_tensor, fill_vec,
           Wq0, bq0, Wk0, bk0, Wv0, bv0, Ws0, bs0, Wb0, g0, be0,
           Wq1, bq1, Wk1, bk1, Wv1, bv1, Ws1, bs1, Wb1, g1, be1):
    src = edge_index[0].astype(jnp.int32)
    dst = edge_index[1].astype(jnp.int32)

    sel = (jnp.arange(256)[:, None] // 16 == jnp.arange(16)[None, :]
           ).astype(jnp.float32)
    exmat = (jnp.arange(16)[:, None] == jnp.arange(128)[None, :] // 16
             ).astype(jnp.float32)
    z128 = jnp.zeros((NP, 128), jnp.float32)
    z16 = jnp.zeros((NP, 16), jnp.float32)
    m128 = jnp.broadcast_to(missing_mask_tensor, (N, 128))

    w0 = jnp.concatenate([Wq0, Wk0, Wv0, Ws0], axis=1)
    b0 = jnp.concatenate([bq0, bk0, bv0, bs0]).reshape(1, 512)
    wba0 = (Wb0[:128, 0] + Wb0[256:, 0]).reshape(1, 128)
    wbb0 = (Wb0[128:256, 0] - Wb0[256:, 0]).reshape(1, 128)

    w1 = jnp.concatenate([Wq1, Wk1, Wv1, Ws1], axis=1)
    b1 = jnp.concatenate([bq1, bk1, bv1, bs1]).reshape(1, 512)
    wba1 = (Wb1[:128, 0] + Wb1[256:, 0]).reshape(1, 128)
    wbb1 = (Wb1[128:256, 0] - Wb1[256:, 0]).reshape(1, 128)

    h = _layer(x_orig, src, dst, w0[:128], w0[128:129].reshape(1, 512), b0,
               wba0, wbb0, g0.reshape(1, 128), be0.reshape(1, 128),
               sel, exmat, z128, z16, m128, fill_vec, True)
    h = _layer(h, src, dst, w1, None, b1,
               wba1, wbb1, g1.reshape(1, 128), be1.reshape(1, 128),
               sel, exmat, z128, z16, None, None, False)
    return h
